# Initial kernel scaffold; baseline (speedup 1.0000x reference)
import jax
import jax.numpy as jnp
from jax.experimental import pallas as pl


def kernel(input_ids, embed_weight, proj_weight, proj_bias):
    raise NotImplementedError("write your pallas kernel here")


# speedup vs baseline: 2.0825x; 2.0825x over previous
import functools

import jax
import jax.numpy as jnp
from jax import lax
from jax.experimental import pallas as pl
from jax.experimental.pallas import tpu as pltpu
from jax.experimental.pallas import tpu_sc as plsc

B_TOKENS = 1024 * 20
D = 1024
V = (1024 + 3) * 32

NC = 2
NS = 16
NW = NC * NS
B_PER_W = B_TOKENS // NW
CHUNK = 40
N_CHUNKS = B_PER_W // CHUNK

MM_BLOCK = 1024


def _sc_gather_body(ids_hbm, table_hbm, out_hbm, idx_v, rows_v, sem):
    wid = lax.axis_index("s") * NC + lax.axis_index("c")
    base = wid * B_PER_W
    pltpu.sync_copy(ids_hbm.at[wid], idx_v)

    def body(j, carry):
        pltpu.async_copy(table_hbm.at[idx_v.at[j]], rows_v, sem).wait()
        pltpu.sync_copy(rows_v, out_hbm.at[pl.ds(base + j * CHUNK, CHUNK)])
        return carry

    lax.fori_loop(0, N_CHUNKS, body, 0)


@functools.partial(
    pl.kernel,
    mesh=plsc.VectorSubcoreMesh(core_axis_name="c", subcore_axis_name="s"),
    out_type=jax.ShapeDtypeStruct((B_TOKENS, D), jnp.float32),
    scratch_types=[
        pltpu.VMEM((N_CHUNKS, CHUNK), jnp.int32),
        pltpu.VMEM((CHUNK, D), jnp.float32),
        pltpu.SemaphoreType.DMA,
    ],
)
def _sc_gather(ids_hbm, table_hbm, out_hbm, idx_v, rows_v, sem):
    _sc_gather_body(ids_hbm, table_hbm, out_hbm, idx_v, rows_v, sem)


def _mm_body(x_ref, w_ref, b_ref, o_ref):
    acc = lax.dot_general(
        x_ref[...], w_ref[...],
        dimension_numbers=(((1,), (1,)), ((), ())),
        preferred_element_type=jnp.float32,
    )
    o_ref[...] = acc + b_ref[...]


def _mm(x, w, b):
    grid = (B_TOKENS // MM_BLOCK,)
    return pl.pallas_call(
        _mm_body,
        grid=grid,
        in_specs=[
            pl.BlockSpec((MM_BLOCK, D), lambda i: (i, 0)),
            pl.BlockSpec((D, D), lambda i: (0, 0)),
            pl.BlockSpec((1, D), lambda i: (0, 0)),
        ],
        out_specs=pl.BlockSpec((MM_BLOCK, D), lambda i: (i, 0)),
        out_shape=jax.ShapeDtypeStruct((B_TOKENS, D), jnp.float32),
    )(x, w, b)


def kernel(input_ids, embed_weight, proj_weight, proj_bias):
    ids = input_ids.reshape(-1).astype(jnp.int32)
    ids3 = ids.reshape(NW, N_CHUNKS, CHUNK)
    gathered = _sc_gather(ids3, embed_weight)
    out = _mm(gathered, proj_weight, proj_bias.reshape(1, D))
    return out.reshape(input_ids.shape[0], input_ids.shape[1], D)


# speedup vs baseline: 2.0848x; 1.0011x over previous
import functools

import jax
import jax.numpy as jnp
from jax import lax
from jax.experimental import pallas as pl
from jax.experimental.pallas import tpu as pltpu
from jax.experimental.pallas import tpu_sc as plsc

B_TOKENS = 1024 * 20
D = 1024
V = (1024 + 3) * 32

NC = 2
NS = 16
NW = NC * NS
B_PER_W = B_TOKENS // NW
CHUNK = 40
N_CHUNKS = B_PER_W // CHUNK

MM_BLOCK = 1024


def _sc_gather_body(ids_hbm, table_hbm, out_hbm, idx_v, rows_v, sem):
    wid = lax.axis_index("s") * NC + lax.axis_index("c")
    base = wid * B_PER_W
    pltpu.sync_copy(ids_hbm.at[wid], idx_v)

    def body(j, carry):
        pltpu.async_copy(table_hbm.at[idx_v.at[j]], rows_v, sem).wait()
        pltpu.sync_copy(rows_v, out_hbm.at[pl.ds(base + j * CHUNK, CHUNK)])
        return carry

    lax.fori_loop(0, N_CHUNKS, body, 0)


@functools.partial(
    pl.kernel,
    mesh=plsc.VectorSubcoreMesh(core_axis_name="c", subcore_axis_name="s"),
    out_type=jax.ShapeDtypeStruct((B_TOKENS, D), jnp.float32),
    scratch_types=[
        pltpu.VMEM((N_CHUNKS, CHUNK), jnp.int32),
        pltpu.VMEM((CHUNK, D), jnp.float32),
        pltpu.SemaphoreType.DMA,
    ],
)
def _sc_gather(ids_hbm, table_hbm, out_hbm, idx_v, rows_v, sem):
    _sc_gather_body(ids_hbm, table_hbm, out_hbm, idx_v, rows_v, sem)


def _mm_body(x_ref, w_ref, b_ref, o_ref):
    acc = lax.dot_general(
        x_ref[...].astype(jnp.bfloat16), w_ref[...].astype(jnp.bfloat16),
        dimension_numbers=(((1,), (1,)), ((), ())),
        preferred_element_type=jnp.float32,
    )
    o_ref[...] = acc + b_ref[...]


def _mm(x, w, b):
    grid = (B_TOKENS // MM_BLOCK,)
    return pl.pallas_call(
        _mm_body,
        grid=grid,
        in_specs=[
            pl.BlockSpec((MM_BLOCK, D), lambda i: (i, 0)),
            pl.BlockSpec((D, D), lambda i: (0, 0)),
            pl.BlockSpec((1, D), lambda i: (0, 0)),
        ],
        out_specs=pl.BlockSpec((MM_BLOCK, D), lambda i: (i, 0)),
        out_shape=jax.ShapeDtypeStruct((B_TOKENS, D), jnp.float32),
    )(x, w, b)


def kernel(input_ids, embed_weight, proj_weight, proj_bias):
    ids = input_ids.reshape(-1).astype(jnp.int32)
    ids3 = ids.reshape(NW, N_CHUNKS, CHUNK)
    gathered = _sc_gather(ids3, embed_weight)
    out = _mm(gathered, proj_weight, proj_bias.reshape(1, D))
    return out.reshape(input_ids.shape[0], input_ids.shape[1], D)


# speedup vs baseline: 4.0691x; 1.9518x over previous
import functools

import jax
import jax.numpy as jnp
from jax import lax
from jax.experimental import pallas as pl
from jax.experimental.pallas import tpu as pltpu
from jax.experimental.pallas import tpu_sc as plsc

B_TOKENS = 1024 * 20
D = 1024
V = (1024 + 3) * 32

NC = 2
NS = 16
NW = NC * NS
B_PER_W = B_TOKENS // NW
CHUNK = 40
N_CHUNKS = B_PER_W // CHUNK

MM_BLOCK = 1024


def _sc_gather_body(ids_hbm, table_hbm, out_hbm, idx_v, rows_v, sem):
    wid = lax.axis_index("s") * NC + lax.axis_index("c")
    base = wid * B_PER_W
    pltpu.sync_copy(ids_hbm.at[wid], idx_v)

    def body(j, carry):
        pltpu.async_copy(table_hbm.at[idx_v.at[j]], rows_v, sem).wait()
        pltpu.sync_copy(rows_v, out_hbm.at[pl.ds(base + j * CHUNK, CHUNK)])
        return carry

    lax.fori_loop(0, N_CHUNKS, body, 0)


@functools.partial(
    pl.kernel,
    mesh=plsc.VectorSubcoreMesh(core_axis_name="c", subcore_axis_name="s"),
    out_type=jax.ShapeDtypeStruct((B_TOKENS, D), jnp.float32),
    scratch_types=[
        pltpu.VMEM((N_CHUNKS, CHUNK), jnp.int32),
        pltpu.VMEM((CHUNK, D), jnp.float32),
        pltpu.SemaphoreType.DMA,
    ],
)
def _sc_gather(ids_hbm, table_hbm, out_hbm, idx_v, rows_v, sem):
    _sc_gather_body(ids_hbm, table_hbm, out_hbm, idx_v, rows_v, sem)


def _mm_body(x_ref, w_ref, b_ref, o_ref):
    acc = lax.dot_general(
        x_ref[...].astype(jnp.bfloat16), w_ref[...].astype(jnp.bfloat16),
        dimension_numbers=(((1,), (1,)), ((), ())),
        preferred_element_type=jnp.float32,
    )
    o_ref[...] = acc + b_ref[...]


def _mm(x, w, b):
    grid = (B_TOKENS // MM_BLOCK,)
    return pl.pallas_call(
        _mm_body,
        grid=grid,
        in_specs=[
            pl.BlockSpec((MM_BLOCK, D), lambda i: (i, 0)),
            pl.BlockSpec((D, D), lambda i: (0, 0)),
            pl.BlockSpec((1, D), lambda i: (0, 0)),
        ],
        out_specs=pl.BlockSpec((MM_BLOCK, D), lambda i: (i, 0)),
        out_shape=jax.ShapeDtypeStruct((B_TOKENS, D), jnp.float32),
    )(x, w, b)


def kernel(input_ids, embed_weight, proj_weight, proj_bias):
    b, l = input_ids.shape
    ids = input_ids.T.reshape(-1).astype(jnp.int32)
    ids3 = ids.reshape(NW, N_CHUNKS, CHUNK)
    gathered = _sc_gather(ids3, embed_weight)
    out = _mm(gathered, proj_weight, proj_bias.reshape(1, D))
    return jnp.swapaxes(out.reshape(l, b, D), 0, 1)


# speedup vs baseline: 4.1654x; 1.0237x over previous
import functools

import jax
import jax.numpy as jnp
from jax import lax
from jax.experimental import pallas as pl
from jax.experimental.pallas import tpu as pltpu
from jax.experimental.pallas import tpu_sc as plsc

B_TOKENS = 1024 * 20
D = 1024

NC = 2
NS = 16
NW = NC * NS

N_PIPE = 4
ROWS_C = B_TOKENS // N_PIPE
B_PER_W = ROWS_C // NW
CHUNK = 40
N_CHUNKS = B_PER_W // CHUNK

MM_BLOCK = 1024
MM_STEPS = ROWS_C // MM_BLOCK


@functools.partial(
    pl.kernel,
    mesh=plsc.VectorSubcoreMesh(core_axis_name="c", subcore_axis_name="s"),
    out_type=jax.ShapeDtypeStruct((ROWS_C, D), jnp.float32),
    scratch_types=[
        pltpu.VMEM((N_CHUNKS, CHUNK), jnp.int32),
        pltpu.VMEM((CHUNK, D), jnp.float32),
        pltpu.SemaphoreType.DMA,
    ],
)
def _sc_gather(ids_hbm, table_hbm, out_hbm, idx_v, rows_v, sem):
    wid = lax.axis_index("s") * NC + lax.axis_index("c")
    base = wid * B_PER_W
    pltpu.sync_copy(ids_hbm.at[wid], idx_v)

    def body(j, carry):
        pltpu.async_copy(table_hbm.at[idx_v.at[j]], rows_v, sem).wait()
        pltpu.sync_copy(rows_v, out_hbm.at[pl.ds(base + j * CHUNK, CHUNK)])
        return carry

    lax.fori_loop(0, N_CHUNKS, body, 0)


def _mm_compute(x_ref, w_ref, b_ref, o_ref):
    acc = lax.dot_general(
        x_ref[...].astype(jnp.bfloat16), w_ref[...].astype(jnp.bfloat16),
        dimension_numbers=(((1,), (1,)), ((), ())),
        preferred_element_type=jnp.float32,
    )
    o_ref[...] = acc + b_ref[...]


def _mm_first(x, w, b):
    return pl.pallas_call(
        _mm_compute,
        grid=(MM_STEPS,),
        in_specs=[
            pl.BlockSpec((MM_BLOCK, D), lambda i: (i, 0)),
            pl.BlockSpec((D, D), lambda i: (0, 0)),
            pl.BlockSpec((1, D), lambda i: (0, 0)),
        ],
        out_specs=pl.BlockSpec((MM_BLOCK, D), lambda i: (i, 0)),
        out_shape=jax.ShapeDtypeStruct((B_TOKENS, D), jnp.float32),
    )(x, w, b)


def _mm_chunk(c, prev, x, w, b):
    def body(prev_ref, x_ref, w_ref, b_ref, o_ref):
        del prev_ref
        _mm_compute(x_ref, w_ref, b_ref, o_ref)

    return pl.pallas_call(
        body,
        grid=(MM_STEPS,),
        in_specs=[
            pl.BlockSpec(memory_space=pltpu.MemorySpace.HBM),
            pl.BlockSpec((MM_BLOCK, D), lambda i: (i, 0)),
            pl.BlockSpec((D, D), lambda i: (0, 0)),
            pl.BlockSpec((1, D), lambda i: (0, 0)),
        ],
        out_specs=pl.BlockSpec(
            (MM_BLOCK, D), lambda i, c=c: (c * MM_STEPS + i, 0)
        ),
        out_shape=jax.ShapeDtypeStruct((B_TOKENS, D), jnp.float32),
        input_output_aliases={0: 0},
    )(prev, x, w, b)


def kernel(input_ids, embed_weight, proj_weight, proj_bias):
    b, l = input_ids.shape
    ids = input_ids.T.reshape(-1).astype(jnp.int32)
    bias2 = proj_bias.reshape(1, D)
    gathered = [
        _sc_gather(
            ids[c * ROWS_C:(c + 1) * ROWS_C].reshape(NW, N_CHUNKS, CHUNK),
            embed_weight,
        )
        for c in range(N_PIPE)
    ]
    out = _mm_first(gathered[0], proj_weight, bias2)
    for c in range(1, N_PIPE):
        out = _mm_chunk(c, out, gathered[c], proj_weight, bias2)
    return jnp.swapaxes(out.reshape(l, b, D), 0, 1)
